# Initial kernel scaffold; baseline (speedup 1.0000x reference)
import jax
import jax.numpy as jnp
from jax.experimental import pallas as pl


def kernel(indices, table):
    raise NotImplementedError("write your pallas kernel here")


# speedup vs baseline: 1.4578x; 1.4578x over previous
import functools

import jax
import jax.numpy as jnp
from jax import lax
from jax.experimental import pallas as pl
from jax.experimental.pallas import tpu as pltpu
from jax.experimental.pallas import tpu_sc as plsc

_NC = 2
_NS = 16
_NW = _NC * _NS

_IDXW = 128
_K = 8


def _embed_call(n_rows, d, k, rows_per_w):
  n_chunks = rows_per_w // k
  mesh = plsc.VectorSubcoreMesh(core_axis_name="c", subcore_axis_name="s")

  @functools.partial(
      pl.kernel,
      mesh=mesh,
      out_type=jax.ShapeDtypeStruct((n_rows, _IDXW, d), jnp.float32),
      compiler_params=pltpu.CompilerParams(use_tc_tiling_on_sc=False),
      scratch_types=[
          pltpu.VMEM((k, _IDXW), jnp.int32),
          pltpu.VMEM((k, _IDXW, d), jnp.float32),
          pltpu.SemaphoreType.DMA,
      ],
  )
  def body(table_hbm, idx_hbm, out_hbm, idx_v, rows_v, sem):
    wid = lax.axis_index("s") * _NC + lax.axis_index("c")
    base = wid * rows_per_w

    @pl.loop(0, n_chunks)
    def _chunk(c):
      row0 = base + c * k
      pltpu.sync_copy(idx_hbm.at[pl.ds(row0, k)], idx_v)
      descs = []
      for j in range(k):
        descs.append(
            pltpu.async_copy(table_hbm.at[idx_v.at[j]], rows_v.at[j], sem))
      for dsc in descs:
        dsc.wait()
      pltpu.sync_copy(rows_v, out_hbm.at[pl.ds(row0, k)])

  return body


def kernel(indices, table):
  batch, seq = indices.shape
  vocab1, d = table.shape
  total = batch * seq
  n_rows = total // _IDXW
  rows_per_w = n_rows // _NW

  shifted = (indices.reshape(total) + 1).astype(jnp.int32)
  idx2d = shifted.reshape(n_rows, _IDXW)

  out = _embed_call(n_rows, d, _K, rows_per_w)(table, idx2d)
  return out.reshape(batch, seq, d)


# speedup vs baseline: 1.4774x; 1.0134x over previous
import functools

import jax
import jax.numpy as jnp
from jax import lax
from jax.experimental import pallas as pl
from jax.experimental.pallas import tpu as pltpu
from jax.experimental.pallas import tpu_sc as plsc

_NC = 2
_NS = 16
_NW = _NC * _NS

_CH = 1024


def _embed_call(total, d, idx_per_w):
  n_chunks = idx_per_w // _CH
  mesh = plsc.VectorSubcoreMesh(core_axis_name="c", subcore_axis_name="s")

  @functools.partial(
      pl.kernel,
      mesh=mesh,
      out_type=jax.ShapeDtypeStruct((total, d), jnp.float32),
      compiler_params=pltpu.CompilerParams(use_tc_tiling_on_sc=False),
      scratch_types=[
          pltpu.VMEM((idx_per_w,), jnp.int32),
          pltpu.VMEM((_CH, d), jnp.float32),
          pltpu.SemaphoreType.DMA,
      ],
  )
  def body(table_hbm, idx_hbm, out_hbm, idx_all, rows_v, sem):
    wid = lax.axis_index("s") * _NC + lax.axis_index("c")
    base = wid * idx_per_w
    pltpu.sync_copy(idx_hbm.at[pl.ds(base, idx_per_w)], idx_all)

    @pl.loop(0, n_chunks)
    def _chunk(c):
      idx_c = idx_all.at[pl.ds(c * _CH, _CH)]
      pltpu.async_copy(table_hbm.at[idx_c], rows_v, sem).wait()
      pltpu.sync_copy(rows_v, out_hbm.at[pl.ds(base + c * _CH, _CH)])

  return body


def kernel(indices, table):
  batch, seq = indices.shape
  vocab1, d = table.shape
  total = batch * seq
  idx_per_w = total // _NW

  shifted = (indices.reshape(total) + 1).astype(jnp.int32)
  out = _embed_call(total, d, idx_per_w)(table, shifted)
  return out.reshape(batch, seq, d)


# speedup vs baseline: 1.5016x; 1.0164x over previous
import functools

import jax
import jax.numpy as jnp
from jax import lax
from jax.experimental import pallas as pl
from jax.experimental.pallas import tpu as pltpu
from jax.experimental.pallas import tpu_sc as plsc

_NC = 2
_NS = 16
_NW = _NC * _NS

_CH = 800
_NBUF = 4


def _embed_call(total, d, idx_per_w):
  n_chunks = idx_per_w // _CH
  mesh = plsc.VectorSubcoreMesh(core_axis_name="c", subcore_axis_name="s")

  @functools.partial(
      pl.kernel,
      mesh=mesh,
      out_type=jax.ShapeDtypeStruct((total, d), jnp.float32),
      compiler_params=pltpu.CompilerParams(use_tc_tiling_on_sc=False),
      scratch_types=[
          pltpu.VMEM((idx_per_w,), jnp.int32),
          pltpu.VMEM((_NBUF, _CH, d), jnp.float32),
          [pltpu.SemaphoreType.DMA] * _NBUF,
          [pltpu.SemaphoreType.DMA] * _NBUF,
      ],
  )
  def body(table_hbm, idx_hbm, out_hbm, idx_all, rows, gsem, wsem):
    wid = lax.axis_index("s") * _NC + lax.axis_index("c")
    base = wid * idx_per_w
    pltpu.sync_copy(idx_hbm.at[pl.ds(base, idx_per_w)], idx_all)

    def idx_c(c):
      return idx_all.at[pl.ds(c * _CH, _CH)]

    def out_c(c):
      return out_hbm.at[pl.ds(base + c * _CH, _CH)]

    def fire(c, b):
      pltpu.async_copy(table_hbm.at[idx_c(c)], rows.at[b], gsem[b])

    def drain(c, b):
      pltpu.make_async_copy(table_hbm.at[idx_c(c)], rows.at[b],
                            gsem[b]).wait()

    def wb_start(c, b):
      pltpu.async_copy(rows.at[b], out_c(c), wsem[b])

    def wb_wait(c, b):
      pltpu.make_async_copy(rows.at[b], out_c(c), wsem[b]).wait()

    fire(0, 0)
    fire(1, 1)
    fire(2, 2)

    drain(0, 0)
    wb_start(0, 0)
    fire(3, 3)

    @pl.loop(0, (n_chunks - 4) // _NBUF)
    def _trip(g):
      for r in range(_NBUF):
        c = g * _NBUF + 1 + r
        b = (1 + r) % _NBUF
        pb = r % _NBUF
        drain(c, b)
        wb_start(c, b)
        wb_wait(c - 1, pb)
        fire(c + 3, pb)

    for c in (n_chunks - 3, n_chunks - 2, n_chunks - 1):
      b = c % _NBUF
      drain(c, b)
      wb_start(c, b)
    for c in (n_chunks - 4, n_chunks - 3, n_chunks - 2, n_chunks - 1):
      wb_wait(c, c % _NBUF)

  return body


def kernel(indices, table):
  batch, seq = indices.shape
  vocab1, d = table.shape
  total = batch * seq
  idx_per_w = total // _NW

  shifted = (indices.reshape(total) + 1).astype(jnp.int32)
  out = _embed_call(total, d, idx_per_w)(table, shifted)
  return out.reshape(batch, seq, d)
